# baseline (speedup 1.0000x reference)
import functools

import jax
import jax.numpy as jnp
from jax import lax
from jax.experimental import pallas as pl
from jax.experimental.pallas import tpu as pltpu
from jax.experimental.pallas import tpu_sc as plsc

B = 16
L = 4096
N = 32768
D_IN = 4
D_MID = 64
D_LOC = 16
D_OUT = 256

NPAD = N + L
TA = 4096
NTA = NPAD // TA

NW = 32
RPW = (B * L) // NW
JB = 128
NJ = RPW // JB


def _mlp_body(feats_ref, ids_ref, W1_ref, b1_ref, hpad_ref, cnt_ref,
              offs_ref, idx_ref):
    i = pl.program_id(0)

    @pl.when(i == 0)
    def _():
        ids = ids_ref[...]
        subl = lax.broadcasted_iota(jnp.int32, (B, 1), 0)
        offcol = jnp.zeros((B, 1), jnp.int32)
        cntcol = jnp.zeros((B, 1), jnp.int32)
        for b in range(B):
            cb = jnp.sum((ids == b).astype(jnp.int32))
            ob = jnp.sum((ids < b).astype(jnp.int32))
            cnt_ref[b] = cb
            offs_ref[b] = ob
            offcol = offcol + jnp.where(subl == b, ob, 0)
            cntcol = cntcol + jnp.where(subl == b, jnp.minimum(cb, L), 0)
        lane = lax.broadcasted_iota(jnp.int32, (B, L), 1)
        idx_ref[...] = jnp.where(lane < cntcol, offcol + lane, N)

    f = feats_ref[...]
    h = jnp.maximum(
        jnp.dot(f, W1_ref[...], preferred_element_type=jnp.float32)
        + b1_ref[...], 0.0)

    @pl.when(i < NTA - 1)
    def _():
        hpad_ref[...] = h

    @pl.when(i == NTA - 1)
    def _():
        hpad_ref[...] = jnp.zeros((TA, D_MID), jnp.float32)


def _sc_pad_body(hpad_hbm, idx_hbm, mfeat_hbm, idxbuf, g0, g1, sem_i,
                 sem_g0, sem_g1, sem_o0, sem_o1):
    c = lax.axis_index("c")
    s = lax.axis_index("s")
    wid = s * 2 + c
    dst0 = wid * RPW

    pltpu.async_copy(idx_hbm.at[pl.ds(wid * NJ, NJ)], idxbuf, sem_i).wait()

    bufs = [g0, g1]
    gsems = [sem_g0, sem_g1]
    osems = [sem_o0, sem_o1]
    ghandles = [None, None]
    ohandles = [None, None]
    ghandles[0] = pltpu.async_copy(hpad_hbm.at[idxbuf.at[0]], g0, sem_g0)
    for j in range(NJ):
        p = j % 2
        q = (j + 1) % 2
        ghandles[p].wait()
        if j + 1 < NJ:
            if ohandles[q] is not None:
                ohandles[q].wait()
            ghandles[q] = pltpu.async_copy(hpad_hbm.at[idxbuf.at[j + 1]],
                                           bufs[q], gsems[q])
        ohandles[p] = pltpu.async_copy(
            bufs[p], mfeat_hbm.at[pl.ds(dst0 + j * JB, JB)], osems[p])
    ohandles[0].wait()
    ohandles[1].wait()


_sc_pad = functools.partial(
    pl.kernel,
    out_type=jax.ShapeDtypeStruct((B * L, D_MID), jnp.float32),
    mesh=plsc.VectorSubcoreMesh(core_axis_name="c", subcore_axis_name="s",
                                num_cores=2, num_subcores=16),
    scratch_types=[
        pltpu.VMEM((NJ, JB), jnp.int32),
        pltpu.VMEM((JB, D_MID), jnp.float32),
        pltpu.VMEM((JB, D_MID), jnp.float32),
        pltpu.SemaphoreType.DMA,
        pltpu.SemaphoreType.DMA,
        pltpu.SemaphoreType.DMA,
        pltpu.SemaphoreType.DMA,
        pltpu.SemaphoreType.DMA,
    ],
    compiler_params=pltpu.CompilerParams(use_tc_tiling_on_sc=False),
)(_sc_pad_body)


def _pool_body(hpad_ref, cnt_ref, offs_ref, W2_ref, b2c_ref, Wfc_ref,
               bfc_ref, out_ref, pooled):
    bi = pl.program_id(0)
    cb = jnp.minimum(cnt_ref[bi], L)
    ob = offs_ref[bi]
    hch = hpad_ref[pl.ds(ob, L), :]

    xT = lax.dot_general(W2_ref[...], hch, (((0,), (1,)), ((), ())),
                         preferred_element_type=jnp.float32) + b2c_ref[...]
    lane_iota = lax.broadcasted_iota(jnp.int32, (1, L), 1)
    xTm = jnp.where(lane_iota < cb, xT, xT[:, 0:1])
    cols = []
    for jj in range(D_LOC):
        prod = xTm * xTm[jj:jj + 1, :]
        cols.append(jnp.max(prod, axis=1, keepdims=True))
    tile = jnp.concatenate(cols, axis=1)
    flat = jnp.concatenate(
        [tile[ii:ii + 1, :] for ii in range(D_LOC)], axis=1)
    flat = jnp.where(cb > 0, flat, jnp.full_like(flat, -1e30))
    pooled[pl.ds(bi, 1), :] = flat

    @pl.when(bi == B - 1)
    def _():
        P = pooled[...]
        pe = jnp.sign(P) * jnp.sqrt(jnp.abs(P) + 1e-8)
        nrm = jnp.sqrt(jnp.sum(pe * pe, axis=1, keepdims=True))
        flatn = pe / (nrm + 1e-12)
        out_ref[...] = jnp.dot(flatn, Wfc_ref[...],
                               preferred_element_type=jnp.float32) + bfc_ref[...]


def kernel(feats, W1, b1, W2, b2, W_fc, b_fc, batch_ids):
    feats_pad = jnp.concatenate(
        [feats, jnp.zeros((NPAD - N, D_IN), jnp.float32)], axis=0)
    ids2d = batch_ids.astype(jnp.int32).reshape(B, N // B)
    b1r = b1.reshape(1, D_MID)
    b2c = b2.reshape(D_LOC, 1)
    bfcr = b_fc.reshape(1, D_OUT)

    hpad, cnt, offs, idx = pl.pallas_call(
        _mlp_body,
        grid=(NTA,),
        in_specs=[
            pl.BlockSpec((TA, D_IN), lambda i: (i, 0)),
            pl.BlockSpec((B, N // B), lambda i: (0, 0)),
            pl.BlockSpec((D_IN, D_MID), lambda i: (0, 0)),
            pl.BlockSpec((1, D_MID), lambda i: (0, 0)),
        ],
        out_specs=[
            pl.BlockSpec((TA, D_MID), lambda i: (i, 0)),
            pl.BlockSpec(memory_space=pltpu.SMEM),
            pl.BlockSpec(memory_space=pltpu.SMEM),
            pl.BlockSpec((B, L), lambda i: (0, 0)),
        ],
        out_shape=[
            jax.ShapeDtypeStruct((NPAD, D_MID), jnp.float32),
            jax.ShapeDtypeStruct((B,), jnp.int32),
            jax.ShapeDtypeStruct((B,), jnp.int32),
            jax.ShapeDtypeStruct((B, L), jnp.int32),
        ],
    )(feats_pad, ids2d, W1, b1r)

    idx2 = idx.reshape(NW * NJ, JB)
    mfeat = _sc_pad(hpad, idx2).reshape(B, L, D_MID)

    out = pl.pallas_call(
        _pool_body,
        grid=(B,),
        in_specs=[
            pl.BlockSpec((NPAD, D_MID), lambda i: (0, 0)),
            pl.BlockSpec(memory_space=pltpu.SMEM),
            pl.BlockSpec(memory_space=pltpu.SMEM),
            pl.BlockSpec((D_MID, D_LOC), lambda i: (0, 0)),
            pl.BlockSpec((D_LOC, 1), lambda i: (0, 0)),
            pl.BlockSpec((D_LOC * D_LOC, D_OUT), lambda i: (0, 0)),
            pl.BlockSpec((1, D_OUT), lambda i: (0, 0)),
        ],
        out_specs=pl.BlockSpec((B, D_OUT), lambda i: (0, 0)),
        out_shape=jax.ShapeDtypeStruct((B, D_OUT), jnp.float32),
        scratch_shapes=[
            pltpu.VMEM((B, D_LOC * D_LOC), jnp.float32),
        ],
        compiler_params=pltpu.CompilerParams(
            vmem_limit_bytes=100 * 1024 * 1024),
    )(hpad, cnt, offs, W2, b2c, W_fc, bfcr)
    return out, mfeat


# speedup vs baseline: 1.2376x; 1.2376x over previous
import functools

import jax
import jax.numpy as jnp
from jax import lax
from jax.experimental import pallas as pl
from jax.experimental.pallas import tpu as pltpu
from jax.experimental.pallas import tpu_sc as plsc

B = 16
L = 4096
N = 32768
D_IN = 4
D_MID = 64
D_LOC = 16
D_OUT = 256

NPAD = N + L
TA = 4096
NTA = NPAD // TA

NW = 32
RPW = (B * L) // NW
JB = 128
NJ = RPW // JB


def _mlp_body(feats_ref, ids_ref, W1_ref, b1_ref, hpad_ref, cnt_ref,
              offs_ref):
    i = pl.program_id(0)

    @pl.when(i == 0)
    def _():
        ids = ids_ref[...]
        for b in range(B):
            cnt_ref[b] = jnp.sum((ids == b).astype(jnp.int32))
            offs_ref[b] = jnp.sum((ids < b).astype(jnp.int32))

    f = feats_ref[...]
    h = jnp.maximum(
        jnp.dot(f, W1_ref[...], preferred_element_type=jnp.float32)
        + b1_ref[...], 0.0)

    @pl.when(i < NTA - 1)
    def _():
        hpad_ref[...] = h

    @pl.when(i == NTA - 1)
    def _():
        hpad_ref[...] = jnp.zeros((TA, D_MID), jnp.float32)


BPC = B // 2


def _sc_pad_body(hpad_hbm, cnt_hbm, offs_hbm, mfeat_hbm, cnts, offss, sem_s,
                 sem_d):
    cid = lax.axis_index("c")
    pltpu.async_copy(cnt_hbm, cnts, sem_s).wait()
    pltpu.async_copy(offs_hbm, offss, sem_s).wait()

    for bl in range(BPC):
        b = bl * 2 + cid
        cb = jnp.minimum(cnts[b], L)
        ob = offss[b]
        dstb = b * L
        run = 0
        for sbit in (4096, 2048, 1024, 512, 256, 128, 64, 32, 16, 8, 4,
                     2, 1):
            @pl.when((cb & sbit) != 0)
            def _(run=run, sbit=sbit, ob=ob, dstb=dstb):
                pltpu.async_copy(hpad_hbm.at[pl.ds(ob + run, sbit)],
                                 mfeat_hbm.at[pl.ds(dstb + run, sbit)],
                                 sem_d)
            run = jnp.where((cb & sbit) != 0, run + sbit, run)
        zb = L - cb
        zrun = 0
        for sbit in (4096, 2048, 1024, 512, 256, 128, 64, 32, 16, 8, 4,
                     2, 1):
            @pl.when((zb & sbit) != 0)
            def _(zrun=zrun, sbit=sbit, cb=cb, dstb=dstb):
                pltpu.async_copy(hpad_hbm.at[pl.ds(N, sbit)],
                                 mfeat_hbm.at[pl.ds(dstb + cb + zrun, sbit)],
                                 sem_d)
            zrun = jnp.where((zb & sbit) != 0, zrun + sbit, zrun)

    pltpu.make_async_copy(
        hpad_hbm.at[pl.ds(0, BPC * L)],
        mfeat_hbm.at[pl.ds(cid * BPC * L, BPC * L)], sem_d).wait()


_sc_pad = functools.partial(
    pl.kernel,
    out_type=jax.ShapeDtypeStruct((B * L, D_MID), jnp.float32),
    mesh=plsc.ScalarSubcoreMesh(axis_name="c", num_cores=2),
    scratch_types=[
        pltpu.SMEM((16,), jnp.int32),
        pltpu.SMEM((16,), jnp.int32),
        pltpu.SemaphoreType.DMA,
        pltpu.SemaphoreType.DMA,
    ],
    compiler_params=pltpu.CompilerParams(use_tc_tiling_on_sc=False),
)(_sc_pad_body)


def _pool_body(hpad_ref, cnt_ref, offs_ref, W2_ref, b2c_ref, Wfc_ref,
               bfc_ref, out_ref, pooled):
    bi = pl.program_id(0)
    cb = jnp.minimum(cnt_ref[bi], L)
    ob = offs_ref[bi]
    hch = hpad_ref[pl.ds(ob, L), :]

    xT = lax.dot_general(W2_ref[...], hch, (((0,), (1,)), ((), ())),
                         preferred_element_type=jnp.float32) + b2c_ref[...]
    lane_iota = lax.broadcasted_iota(jnp.int32, (1, L), 1)
    xTm = jnp.where(lane_iota < cb, xT, xT[:, 0:1])
    cols = []
    for jj in range(D_LOC):
        prod = xTm * xTm[jj:jj + 1, :]
        cols.append(jnp.max(prod, axis=1, keepdims=True))
    tile = jnp.concatenate(cols, axis=1)
    flat = jnp.concatenate(
        [tile[ii:ii + 1, :] for ii in range(D_LOC)], axis=1)
    flat = jnp.where(cb > 0, flat, jnp.full_like(flat, -1e30))
    pooled[pl.ds(bi, 1), :] = flat

    @pl.when(bi == B - 1)
    def _():
        P = pooled[...]
        pe = jnp.sign(P) * jnp.sqrt(jnp.abs(P) + 1e-8)
        nrm = jnp.sqrt(jnp.sum(pe * pe, axis=1, keepdims=True))
        flatn = pe / (nrm + 1e-12)
        out_ref[...] = jnp.dot(flatn, Wfc_ref[...],
                               preferred_element_type=jnp.float32) + bfc_ref[...]


def kernel(feats, W1, b1, W2, b2, W_fc, b_fc, batch_ids):
    feats_pad = jnp.concatenate(
        [feats, jnp.zeros((NPAD - N, D_IN), jnp.float32)], axis=0)
    ids2d = batch_ids.astype(jnp.int32).reshape(B, N // B)
    b1r = b1.reshape(1, D_MID)
    b2c = b2.reshape(D_LOC, 1)
    bfcr = b_fc.reshape(1, D_OUT)

    hpad, cnt, offs = pl.pallas_call(
        _mlp_body,
        grid=(NTA,),
        in_specs=[
            pl.BlockSpec((TA, D_IN), lambda i: (i, 0)),
            pl.BlockSpec((B, N // B), lambda i: (0, 0)),
            pl.BlockSpec((D_IN, D_MID), lambda i: (0, 0)),
            pl.BlockSpec((1, D_MID), lambda i: (0, 0)),
        ],
        out_specs=[
            pl.BlockSpec((TA, D_MID), lambda i: (i, 0)),
            pl.BlockSpec(memory_space=pltpu.SMEM),
            pl.BlockSpec(memory_space=pltpu.SMEM),
        ],
        out_shape=[
            jax.ShapeDtypeStruct((NPAD, D_MID), jnp.float32),
            jax.ShapeDtypeStruct((B,), jnp.int32),
            jax.ShapeDtypeStruct((B,), jnp.int32),
        ],
    )(feats_pad, ids2d, W1, b1r)

    mfeat = _sc_pad(hpad, cnt, offs).reshape(B, L, D_MID)

    out = pl.pallas_call(
        _pool_body,
        grid=(B,),
        in_specs=[
            pl.BlockSpec((NPAD, D_MID), lambda i: (0, 0)),
            pl.BlockSpec(memory_space=pltpu.SMEM),
            pl.BlockSpec(memory_space=pltpu.SMEM),
            pl.BlockSpec((D_MID, D_LOC), lambda i: (0, 0)),
            pl.BlockSpec((D_LOC, 1), lambda i: (0, 0)),
            pl.BlockSpec((D_LOC * D_LOC, D_OUT), lambda i: (0, 0)),
            pl.BlockSpec((1, D_OUT), lambda i: (0, 0)),
        ],
        out_specs=pl.BlockSpec((B, D_OUT), lambda i: (0, 0)),
        out_shape=jax.ShapeDtypeStruct((B, D_OUT), jnp.float32),
        scratch_shapes=[
            pltpu.VMEM((B, D_LOC * D_LOC), jnp.float32),
        ],
        compiler_params=pltpu.CompilerParams(
            vmem_limit_bytes=100 * 1024 * 1024),
    )(hpad, cnt, offs, W2, b2c, W_fc, bfcr)
    return out, mfeat


# speedup vs baseline: 10.4160x; 8.4161x over previous
import jax
import jax.numpy as jnp
from jax import lax
from jax.experimental import pallas as pl
from jax.experimental.pallas import tpu as pltpu

B = 16
L = 4096
N = 32768
D_IN = 4
D_MID = 64
D_LOC = 16
D_OUT = 256

NPAD = N + L
TA = 12288
NTA = NPAD // TA
GRID = NTA + B


def _body(feats_ref, ids_ref, W1_ref, b1_ref, W2_ref, b2c_ref, Wfc_ref,
          bfc_ref, out_ref, mfeat_ref, hbuf, pooled, cnt, offs):
    i = pl.program_id(0)

    @pl.when(i == 0)
    def _():
        ids = ids_ref[...]
        for b in range(B):
            cnt[b] = jnp.sum((ids == b).astype(jnp.int32))
            offs[b] = jnp.sum((ids < b).astype(jnp.int32))

    @pl.when(i < NTA)
    def _():
        f = feats_ref[...]
        h = jnp.maximum(
            jnp.dot(f, W1_ref[...], preferred_element_type=jnp.float32)
            + b1_ref[...], 0.0)
        hbuf[pl.ds(i * TA, TA), :] = h

    @pl.when(i >= NTA)
    def _():
        b = i - NTA
        cb = jnp.minimum(cnt[b], L)
        ob = offs[b]
        hch = hbuf[pl.ds(ob, L), :]
        row_iota = lax.broadcasted_iota(jnp.int32, (L, 1), 0)
        mfeat_ref[0] = jnp.where(row_iota < cb, hch, 0.0)

        xT = lax.dot_general(W2_ref[...], hch, (((0,), (1,)), ((), ())),
                             preferred_element_type=jnp.float32) + b2c_ref[...]
        lane_iota = lax.broadcasted_iota(jnp.int32, (1, L), 1)
        xb = xT.astype(jnp.bfloat16)
        xbm = jnp.where(lane_iota < cb, xb, xb[:, 0:1])
        cols = []
        for jj in range(D_LOC):
            prod = xbm * xbm[jj:jj + 1, :]
            cols.append(jnp.max(prod, axis=1, keepdims=True)
                        .astype(jnp.float32))
        tile = jnp.concatenate(cols, axis=1)
        flat = jnp.concatenate(
            [tile[ii:ii + 1, :] for ii in range(D_LOC)], axis=1)
        flat = jnp.where(cb > 0, flat, jnp.full_like(flat, -1e30))
        pooled[pl.ds(b, 1), :] = flat

    @pl.when(i == GRID - 1)
    def _():
        P = pooled[...]
        pe = jnp.sign(P) * jnp.sqrt(jnp.abs(P) + 1e-8)
        nrm = jnp.sqrt(jnp.sum(pe * pe, axis=1, keepdims=True))
        flatn = pe / (nrm + 1e-12)
        out_ref[...] = jnp.dot(flatn, Wfc_ref[...],
                               preferred_element_type=jnp.float32) + bfc_ref[...]


def kernel(feats, W1, b1, W2, b2, W_fc, b_fc, batch_ids):
    feats_pad = jnp.concatenate(
        [feats, jnp.zeros((NPAD - N, D_IN), jnp.float32)], axis=0)
    ids2d = batch_ids.astype(jnp.int32).reshape(B, N // B)
    b1r = b1.reshape(1, D_MID)
    b2c = b2.reshape(D_LOC, 1)
    bfcr = b_fc.reshape(1, D_OUT)

    out, mfeat = pl.pallas_call(
        _body,
        grid=(GRID,),
        in_specs=[
            pl.BlockSpec((TA, D_IN), lambda i: (jnp.minimum(i, NTA - 1), 0)),
            pl.BlockSpec((B, N // B), lambda i: (0, 0)),
            pl.BlockSpec((D_IN, D_MID), lambda i: (0, 0)),
            pl.BlockSpec((1, D_MID), lambda i: (0, 0)),
            pl.BlockSpec((D_MID, D_LOC), lambda i: (0, 0)),
            pl.BlockSpec((D_LOC, 1), lambda i: (0, 0)),
            pl.BlockSpec((D_LOC * D_LOC, D_OUT), lambda i: (0, 0)),
            pl.BlockSpec((1, D_OUT), lambda i: (0, 0)),
        ],
        out_specs=[
            pl.BlockSpec((B, D_OUT), lambda i: (0, 0)),
            pl.BlockSpec(
                (1, L, D_MID),
                lambda i: (jnp.maximum(i - NTA, 0), 0, 0)),
        ],
        out_shape=[
            jax.ShapeDtypeStruct((B, D_OUT), jnp.float32),
            jax.ShapeDtypeStruct((B, L, D_MID), jnp.float32),
        ],
        scratch_shapes=[
            pltpu.VMEM((NPAD, D_MID), jnp.float32),
            pltpu.VMEM((B, D_LOC * D_LOC), jnp.float32),
            pltpu.SMEM((B,), jnp.int32),
            pltpu.SMEM((B,), jnp.int32),
        ],
        compiler_params=pltpu.CompilerParams(
            vmem_limit_bytes=100 * 1024 * 1024),
    )(feats_pad, ids2d, W1, b1r, W2, b2c, W_fc, bfcr)
    return out, mfeat


# speedup vs baseline: 12.5761x; 1.2074x over previous
import jax
import jax.numpy as jnp
from jax import lax
from jax.experimental import pallas as pl
from jax.experimental.pallas import tpu as pltpu

B = 16
L = 4096
N = 32768
D_IN = 4
D_MID = 64
D_LOC = 16
D_OUT = 256

NPAD = N + L
TA = 8192
NTA = N // TA
GRID = NTA + B


def _body(feats_ref, ids_ref, W1_ref, b1_ref, W2_ref, b2c_ref, Wfc_ref,
          bfc_ref, out_ref, mfeat_ref, hbuf, pooled, cnt, offs):
    i = pl.program_id(0)

    @pl.when(i == 0)
    def _():
        ids = ids_ref[...]
        for b in range(B):
            cnt[b] = jnp.sum((ids == b).astype(jnp.int32))
            offs[b] = jnp.sum((ids < b).astype(jnp.int32))

    @pl.when(i < NTA)
    def _():
        f = feats_ref[...]
        h = jnp.maximum(
            jnp.dot(f, W1_ref[...], preferred_element_type=jnp.float32)
            + b1_ref[...], 0.0)
        hbuf[pl.ds(i * TA, TA), :] = h

    @pl.when(i >= NTA)
    def _():
        b = i - NTA
        cb = jnp.minimum(cnt[b], L)
        ob = offs[b]
        hch = hbuf[pl.ds(ob, L), :]
        row_iota = lax.broadcasted_iota(jnp.int32, (L, 1), 0)
        mfeat_ref[0] = jnp.where(row_iota < cb, hch, 0.0)

        xT = lax.dot_general(W2_ref[...], hch, (((0,), (1,)), ((), ())),
                             preferred_element_type=jnp.float32) + b2c_ref[...]
        lane_iota = lax.broadcasted_iota(jnp.int32, (1, L), 1)
        xb = xT.astype(jnp.bfloat16)
        xbm = jnp.where(lane_iota < cb, xb, xb[:, 0:1])
        cols = []
        for jj in range(D_LOC):
            prod = xbm * xbm[jj:jj + 1, :]
            cols.append(jnp.max(prod, axis=1, keepdims=True)
                        .astype(jnp.float32))
        tile = jnp.concatenate(cols, axis=1)
        flat = jnp.concatenate(
            [tile[ii:ii + 1, :] for ii in range(D_LOC)], axis=1)
        flat = jnp.where(cb > 0, flat, jnp.full_like(flat, -1e30))
        pooled[pl.ds(b, 1), :] = flat

    @pl.when(i == GRID - 1)
    def _():
        P = pooled[...]
        pe = jnp.sign(P) * jnp.sqrt(jnp.abs(P) + 1e-8)
        nrm = jnp.sqrt(jnp.sum(pe * pe, axis=1, keepdims=True))
        flatn = pe / (nrm + 1e-12)
        out_ref[...] = jnp.dot(flatn, Wfc_ref[...],
                               preferred_element_type=jnp.float32) + bfc_ref[...]


def kernel(feats, W1, b1, W2, b2, W_fc, b_fc, batch_ids):
    ids2d = batch_ids.astype(jnp.int32).reshape(B, N // B)
    b1r = b1.reshape(1, D_MID)
    b2c = b2.reshape(D_LOC, 1)
    bfcr = b_fc.reshape(1, D_OUT)

    out, mfeat = pl.pallas_call(
        _body,
        grid=(GRID,),
        in_specs=[
            pl.BlockSpec((TA, D_IN), lambda i: (jnp.minimum(i, NTA - 1), 0)),
            pl.BlockSpec((B, N // B), lambda i: (0, 0)),
            pl.BlockSpec((D_IN, D_MID), lambda i: (0, 0)),
            pl.BlockSpec((1, D_MID), lambda i: (0, 0)),
            pl.BlockSpec((D_MID, D_LOC), lambda i: (0, 0)),
            pl.BlockSpec((D_LOC, 1), lambda i: (0, 0)),
            pl.BlockSpec((D_LOC * D_LOC, D_OUT), lambda i: (0, 0)),
            pl.BlockSpec((1, D_OUT), lambda i: (0, 0)),
        ],
        out_specs=[
            pl.BlockSpec((B, D_OUT), lambda i: (0, 0)),
            pl.BlockSpec(
                (1, L, D_MID),
                lambda i: (jnp.maximum(i - NTA, 0), 0, 0)),
        ],
        out_shape=[
            jax.ShapeDtypeStruct((B, D_OUT), jnp.float32),
            jax.ShapeDtypeStruct((B, L, D_MID), jnp.float32),
        ],
        scratch_shapes=[
            pltpu.VMEM((NPAD, D_MID), jnp.float32),
            pltpu.VMEM((B, D_LOC * D_LOC), jnp.float32),
            pltpu.SMEM((B,), jnp.int32),
            pltpu.SMEM((B,), jnp.int32),
        ],
        compiler_params=pltpu.CompilerParams(
            vmem_limit_bytes=100 * 1024 * 1024),
    )(feats, ids2d, W1, b1r, W2, b2c, W_fc, bfcr)
    return out, mfeat


# speedup vs baseline: 24.9203x; 1.9816x over previous
import jax
import jax.numpy as jnp
from jax import lax
from jax.experimental import pallas as pl
from jax.experimental.pallas import tpu as pltpu

B = 16
L = 4096
N = 32768
D_IN = 4
D_MID = 64
D_LOC = 16
D_OUT = 256

NPAD = N + L
TA = 8192
NTA = N // TA
GRID = NTA + B


def _body(feats_ref, ids_ref, W1_ref, b1_ref, W2_ref, b2c_ref, Wfc_ref,
          bfc_ref, out_ref, mfeat_ref, hbuf, pooled, cnt, offs):
    i = pl.program_id(0)

    @pl.when(i == 0)
    def _():
        ids = ids_ref[...]
        for b in range(B):
            cnt[b] = jnp.sum((ids == b).astype(jnp.int32))
            offs[b] = jnp.sum((ids < b).astype(jnp.int32))

    @pl.when(i < NTA)
    def _():
        f = feats_ref[...]
        h = jnp.maximum(
            lax.dot_general(f, W1_ref[...], (((0,), (0,)), ((), ())),
                            preferred_element_type=jnp.float32)
            + b1_ref[...], 0.0)
        hbuf[pl.ds(i * TA, TA), :] = h

    @pl.when(i >= NTA)
    def _():
        b = i - NTA
        cb = jnp.minimum(cnt[b], L)
        ob = offs[b]
        hch = hbuf[pl.ds(ob, L), :]
        hchT = jnp.transpose(hch, (1, 0))
        lane_iota = lax.broadcasted_iota(jnp.int32, (1, L), 1)
        mfeat_ref[0] = jnp.where(lane_iota < cb, hchT, 0.0)

        xT = lax.dot_general(W2_ref[...], hchT, (((0,), (0,)), ((), ())),
                             preferred_element_type=jnp.float32) + b2c_ref[...]
        xb = xT.astype(jnp.bfloat16)
        xbm = jnp.where(lane_iota < cb, xb, xb[:, 0:1])
        cols = []
        for jj in range(D_LOC):
            prod = xbm * xbm[jj:jj + 1, :]
            cols.append(jnp.max(prod, axis=1, keepdims=True)
                        .astype(jnp.float32))
        tile = jnp.concatenate(cols, axis=1)
        flat = jnp.concatenate(
            [tile[ii:ii + 1, :] for ii in range(D_LOC)], axis=1)
        flat = jnp.where(cb > 0, flat, jnp.full_like(flat, -1e30))
        pooled[pl.ds(b, 1), :] = flat

    @pl.when(i == GRID - 1)
    def _():
        P = pooled[...]
        pe = jnp.sign(P) * jnp.sqrt(jnp.abs(P) + 1e-8)
        nrm = jnp.sqrt(jnp.sum(pe * pe, axis=1, keepdims=True))
        flatn = pe / (nrm + 1e-12)
        out_ref[...] = jnp.dot(flatn, Wfc_ref[...],
                               preferred_element_type=jnp.float32) + bfc_ref[...]


def kernel(feats, W1, b1, W2, b2, W_fc, b_fc, batch_ids):
    fT = feats.T
    ids2d = batch_ids.astype(jnp.int32).reshape(B, N // B)
    b1r = b1.reshape(1, D_MID)
    b2c = b2.reshape(D_LOC, 1)
    bfcr = b_fc.reshape(1, D_OUT)

    out, mfeat = pl.pallas_call(
        _body,
        grid=(GRID,),
        in_specs=[
            pl.BlockSpec((D_IN, TA), lambda i: (0, jnp.minimum(i, NTA - 1))),
            pl.BlockSpec((B, N // B), lambda i: (0, 0)),
            pl.BlockSpec((D_IN, D_MID), lambda i: (0, 0)),
            pl.BlockSpec((1, D_MID), lambda i: (0, 0)),
            pl.BlockSpec((D_MID, D_LOC), lambda i: (0, 0)),
            pl.BlockSpec((D_LOC, 1), lambda i: (0, 0)),
            pl.BlockSpec((D_LOC * D_LOC, D_OUT), lambda i: (0, 0)),
            pl.BlockSpec((1, D_OUT), lambda i: (0, 0)),
        ],
        out_specs=[
            pl.BlockSpec((B, D_OUT), lambda i: (0, 0)),
            pl.BlockSpec(
                (1, D_MID, L),
                lambda i: (jnp.maximum(i - NTA, 0), 0, 0)),
        ],
        out_shape=[
            jax.ShapeDtypeStruct((B, D_OUT), jnp.float32),
            jax.ShapeDtypeStruct((B, D_MID, L), jnp.float32),
        ],
        scratch_shapes=[
            pltpu.VMEM((NPAD, D_MID), jnp.float32),
            pltpu.VMEM((B, D_LOC * D_LOC), jnp.float32),
            pltpu.SMEM((B,), jnp.int32),
            pltpu.SMEM((B,), jnp.int32),
        ],
        compiler_params=pltpu.CompilerParams(
            vmem_limit_bytes=100 * 1024 * 1024),
    )(fT, ids2d, W1, b1r, W2, b2c, W_fc, bfcr)
    return out, jnp.swapaxes(mfeat, 1, 2)
